# baseline (speedup 1.0000x reference)
import jax
import jax.numpy as jnp
from jax.experimental import pallas as pl
from jax.experimental.pallas import tpu as pltpu

_LANE = 128
_BN = 4


def _dice_body(x_ref, t_ref, o_ref):
    i = pl.program_id(0)

    @pl.when(i == 0)
    def _init():
        o_ref[...] = jnp.zeros_like(o_ref)

    bn, C, H, W = x_ref.shape
    rows = bn * (C - 1) * H
    x = x_ref[:, 1:, :, :].reshape(rows, W).astype(jnp.float32)
    t = t_ref[:, 1:, :, :].reshape(rows, W).astype(jnp.float32)

    s = 0.5 * jnp.tanh(0.5 * x) + 0.5
    pi = (s * t).reshape(rows // 8, 8, W).sum(axis=0)
    pd = (s + t).reshape(rows // 8, 8, W).sum(axis=0)

    acc_i = pi[:, :_LANE]
    acc_d = pd[:, :_LANE]
    for k in range(1, W // _LANE):
        acc_i = acc_i + pi[:, k * _LANE:(k + 1) * _LANE]
        acc_d = acc_d + pd[:, k * _LANE:(k + 1) * _LANE]

    o_ref[0] += acc_i
    o_ref[1] += acc_d


@jax.jit
def kernel(inputs, targets):
    N, C, H, W = inputs.shape
    bn = _BN if N % _BN == 0 else N
    ni = N // bn

    spec = pl.BlockSpec((bn, C, H, W), lambda i: (i, 0, 0, 0))

    out = pl.pallas_call(
        _dice_body,
        out_shape=jax.ShapeDtypeStruct((2, 8, _LANE), jnp.float32),
        grid_spec=pltpu.PrefetchScalarGridSpec(
            num_scalar_prefetch=0,
            grid=(ni,),
            in_specs=[spec, spec],
            out_specs=pl.BlockSpec((2, 8, _LANE), lambda i: (0, 0, 0)),
        ),
        compiler_params=pltpu.CompilerParams(
            dimension_semantics=("arbitrary",),
            vmem_limit_bytes=48 * 1024 * 1024),
    )(inputs, targets)

    sums = jnp.sum(out.reshape(2, 8 * _LANE), axis=1)
    one = jnp.float32(1.0)
    return (2.0 * sums[0] + one) / (sums[1] + one)
